# baseline (speedup 1.0000x reference)
import functools

import jax
import jax.numpy as jnp
from jax import lax
from jax.experimental import pallas as pl
from jax.experimental.pallas import tpu as pltpu
from jax.experimental.pallas import tpu_sc as plsc

NUM_FACTORS = 16
ROWS_PER_BLOCK = 8
BATCH = 16384
L = 16
NC, NS = 2, 16
NW = NC * NS
BPW = BATCH // NW
CHUNK = 128
NCHUNK = BPW // CHUNK
BLK_W = ROWS_PER_BLOCK * NUM_FACTORS


def _build():
    mesh = plsc.VectorSubcoreMesh(core_axis_name="c", subcore_axis_name="s")

    @functools.partial(
        pl.kernel,
        mesh=mesh,
        compiler_params=pltpu.CompilerParams(needs_layout_passes=False),
        out_type=jax.ShapeDtypeStruct((BATCH,), jnp.float32),
        scratch_types=[
            pltpu.VMEM((NCHUNK, CHUNK), jnp.int32),
            pltpu.VMEM((NCHUNK, CHUNK), jnp.int32),
            pltpu.VMEM((NCHUNK, CHUNK), jnp.int32),
            pltpu.VMEM((NCHUNK, CHUNK), jnp.int32),
            pltpu.VMEM((CHUNK, BLK_W), jnp.float32),
            pltpu.VMEM((CHUNK, BLK_W), jnp.float32),
            pltpu.VMEM((BPW,), jnp.float32),
            pltpu.SemaphoreType.DMA,
        ],
    )
    def mf_forward(uid_hbm, eid_hbm, utab_hbm, etab_hbm, out_hbm,
                   uid_v, eid_v, ubx_v, ebx_v, u_v, e_v, o_v, sem):
        wid = lax.axis_index("s") * NC + lax.axis_index("c")
        base = wid * BPW
        row0 = wid * NCHUNK

        pltpu.sync_copy(uid_hbm.at[pl.ds(row0, NCHUNK)], uid_v)
        pltpu.sync_copy(eid_hbm.at[pl.ds(row0, NCHUNK)], eid_v)

        for j in range(NCHUNK):
            for o in range(0, CHUNK, L):
                ubx_v[j, pl.ds(o, L)] = jax.lax.shift_right_logical(
                    uid_v[j, pl.ds(o, L)], 3)
                ebx_v[j, pl.ds(o, L)] = jax.lax.shift_right_logical(
                    eid_v[j, pl.ds(o, L)], 3)

        iota = lax.iota(jnp.int32, L)

        for j in range(NCHUNK):
            cu = pltpu.async_copy(utab_hbm.at[ubx_v.at[j]], u_v, sem)
            ce = pltpu.async_copy(etab_hbm.at[ebx_v.at[j]], e_v, sem)
            cu.wait()
            ce.wait()
            for g in range(CHUNK // L):
                rows = g * L + iota
                usub = (uid_v[j, pl.ds(g * L, L)] & 7) * NUM_FACTORS
                esub = (eid_v[j, pl.ds(g * L, L)] & 7) * NUM_FACTORS
                acc = jnp.zeros((L,), jnp.float32)
                for c in range(NUM_FACTORS):
                    u = plsc.load_gather(u_v, [rows, usub + c])
                    e = plsc.load_gather(e_v, [rows, esub + c])
                    acc = acc + u * e
                o_v[pl.ds(j * CHUNK + g * L, L)] = acc

        pltpu.sync_copy(o_v, out_hbm.at[pl.ds(base, BPW)])

    return mf_forward


_KERNEL = _build()


def kernel(user_id, event_id, user_table, event_table):
    uid2 = user_id.reshape(NW * NCHUNK, CHUNK)
    eid2 = event_id.reshape(NW * NCHUNK, CHUNK)
    utab2 = user_table.reshape(-1, BLK_W)
    etab2 = event_table.reshape(-1, BLK_W)
    return _KERNEL(uid2, eid2, utab2, etab2)


# speedup vs baseline: 1.4444x; 1.4444x over previous
import functools

import jax
import jax.numpy as jnp
from jax import lax
from jax.experimental import pallas as pl
from jax.experimental.pallas import tpu as pltpu
from jax.experimental.pallas import tpu_sc as plsc

NUM_FACTORS = 16
BATCH = 16384
L = 16
NC, NS = 2, 16
NW = NC * NS
BPW = BATCH // NW
CHUNK = 128
NCHUNK = BPW // CHUNK


def _build():
    mesh = plsc.VectorSubcoreMesh(core_axis_name="c", subcore_axis_name="s")

    @functools.partial(
        pl.kernel,
        mesh=mesh,
        compiler_params=pltpu.CompilerParams(needs_layout_passes=False),
        out_type=jax.ShapeDtypeStruct((BATCH,), jnp.float32),
        scratch_types=[
            pltpu.VMEM((NCHUNK, CHUNK), jnp.int32),
            pltpu.VMEM((NCHUNK, CHUNK), jnp.int32),
            pltpu.VMEM((CHUNK, NUM_FACTORS), jnp.float32),
            pltpu.VMEM((CHUNK, NUM_FACTORS), jnp.float32),
            pltpu.VMEM((BPW,), jnp.float32),
            pltpu.SemaphoreType.DMA,
        ],
    )
    def mf_forward(uid_hbm, eid_hbm, utab_hbm, etab_hbm, out_hbm,
                   uid_v, eid_v, u_s, e_s, o_v, sem):
        wid = lax.axis_index("s") * NC + lax.axis_index("c")
        base = wid * BPW
        row0 = wid * NCHUNK

        pltpu.sync_copy(uid_hbm.at[pl.ds(row0, NCHUNK)], uid_v)
        pltpu.sync_copy(eid_hbm.at[pl.ds(row0, NCHUNK)], eid_v)

        iota = lax.iota(jnp.int32, L)

        for j in range(NCHUNK):
            def fetch(g, carry, j=j):
                uvec = uid_v[j, pl.ds(g * L, L)]
                evec = eid_v[j, pl.ds(g * L, L)]
                copies = []
                for k in range(L):
                    p = g * L + k
                    copies.append(pltpu.async_copy(
                        utab_hbm.at[uvec[k]], u_s.at[p], sem))
                    copies.append(pltpu.async_copy(
                        etab_hbm.at[evec[k]], e_s.at[p], sem))
                for cp in copies:
                    cp.wait()
                return carry

            lax.fori_loop(0, CHUNK // L, fetch, 0)

            for g in range(CHUNK // L):
                rows = g * L + iota
                acc = jnp.zeros((L,), jnp.float32)
                for c in range(NUM_FACTORS):
                    cvec = jnp.full((L,), c, jnp.int32)
                    u = plsc.load_gather(u_s, [rows, cvec])
                    e = plsc.load_gather(e_s, [rows, cvec])
                    acc = acc + u * e
                o_v[pl.ds(j * CHUNK + g * L, L)] = acc

        pltpu.sync_copy(o_v, out_hbm.at[pl.ds(base, BPW)])

    return mf_forward


_KERNEL = _build()


def kernel(user_id, event_id, user_table, event_table):
    uid2 = user_id.reshape(NW * NCHUNK, CHUNK)
    eid2 = event_id.reshape(NW * NCHUNK, CHUNK)
    return _KERNEL(uid2, eid2, user_table, event_table)


# speedup vs baseline: 1.4864x; 1.0291x over previous
import functools

import jax
import jax.numpy as jnp
from jax import lax
from jax.experimental import pallas as pl
from jax.experimental.pallas import tpu as pltpu
from jax.experimental.pallas import tpu_sc as plsc

NUM_FACTORS = 16
BATCH = 16384
L = 16
NC, NS = 2, 16
NW = NC * NS
BPW = BATCH // NW
CHUNK = 128
NCHUNK = BPW // CHUNK


def _build():
    mesh = plsc.VectorSubcoreMesh(core_axis_name="c", subcore_axis_name="s")

    @functools.partial(
        pl.kernel,
        mesh=mesh,
        compiler_params=pltpu.CompilerParams(needs_layout_passes=False),
        out_type=jax.ShapeDtypeStruct((BATCH,), jnp.float32),
        scratch_types=[
            pltpu.VMEM((NCHUNK, CHUNK), jnp.int32),
            pltpu.VMEM((NCHUNK, CHUNK), jnp.int32),
            pltpu.VMEM((CHUNK, NUM_FACTORS), jnp.float32),
            pltpu.VMEM((CHUNK, NUM_FACTORS), jnp.float32),
            pltpu.VMEM((BPW,), jnp.float32),
            pltpu.SemaphoreType.DMA,
        ],
    )
    def mf_forward(uid_hbm, eid_hbm, utab_hbm, etab_hbm, out_hbm,
                   uid_v, eid_v, u_s, e_s, o_v, sem):
        wid = lax.axis_index("s") * NC + lax.axis_index("c")
        base = wid * BPW
        row0 = wid * NCHUNK

        pltpu.sync_copy(uid_hbm.at[pl.ds(row0, NCHUNK)], uid_v)
        pltpu.sync_copy(eid_hbm.at[pl.ds(row0, NCHUNK)], eid_v)

        iota = lax.iota(jnp.int32, L)

        for j in range(NCHUNK):
            def fire(g, carry, j=j):
                uvec = uid_v[j, pl.ds(g * L, L)]
                evec = eid_v[j, pl.ds(g * L, L)]
                for k in range(L):
                    p = g * L + k
                    pltpu.async_copy(utab_hbm.at[uvec[k]], u_s.at[p], sem)
                    pltpu.async_copy(etab_hbm.at[evec[k]], e_s.at[p], sem)
                return carry

            lax.fori_loop(0, CHUNK // L, fire, 0)
            def drain(g, carry, j=j):
                for k in range(L):
                    p = g * L + k
                    pltpu.make_async_copy(
                        utab_hbm.at[0], u_s.at[p], sem).wait()
                    pltpu.make_async_copy(
                        etab_hbm.at[0], e_s.at[p], sem).wait()
                return carry

            lax.fori_loop(0, CHUNK // L, drain, 0)

            for g in range(CHUNK // L):
                rows = g * L + iota
                acc = jnp.zeros((L,), jnp.float32)
                for c in range(NUM_FACTORS):
                    cvec = jnp.full((L,), c, jnp.int32)
                    u = plsc.load_gather(u_s, [rows, cvec])
                    e = plsc.load_gather(e_s, [rows, cvec])
                    acc = acc + u * e
                o_v[pl.ds(j * CHUNK + g * L, L)] = acc

        pltpu.sync_copy(o_v, out_hbm.at[pl.ds(base, BPW)])

    return mf_forward


_KERNEL = _build()


def kernel(user_id, event_id, user_table, event_table):
    uid2 = user_id.reshape(NW * NCHUNK, CHUNK)
    eid2 = event_id.reshape(NW * NCHUNK, CHUNK)
    return _KERNEL(uid2, eid2, user_table, event_table)
